# Initial kernel scaffold; baseline (speedup 1.0000x reference)
import jax
import jax.numpy as jnp
from jax.experimental import pallas as pl


def kernel(x, table):
    raise NotImplementedError("write your pallas kernel here")


# speedup vs baseline: 30.6931x; 30.6931x over previous
import jax
import jax.numpy as jnp
from jax.experimental import pallas as pl
from jax.experimental.pallas import tpu as pltpu

V = 4
D = 100
R, C0 = 16384, 200

BJ = 8
BI = 2048
GJ = C0 // BJ
GI = R // BI


def _select_body(xt_ref, table_ref, out_ref):
    xb = xt_ref[...][None, :, :]
    t = table_ref[...]
    c0 = t[0, :][:, None, None]
    c1 = t[1, :][:, None, None]
    c2 = t[2, :][:, None, None]
    c3 = t[3, :][:, None, None]
    lo = jnp.where(xb == 1, c1, c0)
    hi = jnp.where(xb == 3, c3, c2)
    out_ref[...] = jnp.where(xb < 2, lo, hi)


def kernel(x, table):
    xt = jnp.transpose(x.astype(jnp.int32), (1, 0))
    o3 = pl.pallas_call(
        _select_body,
        grid=(GJ, GI),
        in_specs=[
            pl.BlockSpec((BJ, BI), lambda j, i: (j, i)),
            pl.BlockSpec((V, D), lambda j, i: (0, 0)),
        ],
        out_specs=pl.BlockSpec((D, BJ, BI), lambda j, i: (0, j, i)),
        out_shape=jax.ShapeDtypeStruct((D, C0, R), jnp.float32),
    )(xt, table.astype(jnp.float32))
    return jnp.transpose(o3, (2, 1, 0))


# speedup vs baseline: 34.0153x; 1.1082x over previous
import jax
import jax.numpy as jnp
from jax.experimental import pallas as pl
from jax.experimental.pallas import tpu as pltpu

V = 4
D = 100
R, C0 = 16384, 200

BJ = 8
BI = 4096
GJ = C0 // BJ
GI = R // BI


def _select_body(xt_ref, table_ref, out_ref):
    xb = xt_ref[...][None, :, :]
    t = table_ref[...]
    c0 = t[0, :][:, None, None]
    c1 = t[1, :][:, None, None]
    c2 = t[2, :][:, None, None]
    c3 = t[3, :][:, None, None]
    lo = jnp.where(xb == 1, c1, c0)
    hi = jnp.where(xb == 3, c3, c2)
    out_ref[...] = jnp.where(xb < 2, lo, hi)


def kernel(x, table):
    xt = jnp.transpose(x.astype(jnp.int32), (1, 0))
    o3 = pl.pallas_call(
        _select_body,
        grid=(GJ, GI),
        in_specs=[
            pl.BlockSpec((BJ, BI), lambda j, i: (j, i)),
            pl.BlockSpec((V, D), lambda j, i: (0, 0)),
        ],
        out_specs=pl.BlockSpec((D, BJ, BI), lambda j, i: (0, j, i)),
        out_shape=jax.ShapeDtypeStruct((D, C0, R), jnp.float32),
    )(xt, table.astype(jnp.float32))
    return jnp.transpose(o3, (2, 1, 0))


# speedup vs baseline: 34.3076x; 1.0086x over previous
import jax
import jax.numpy as jnp
from jax.experimental import pallas as pl

V = 4
D = 100
R, C0 = 16384, 200

BJ = 8
BI = 8192
DD = 10
GJ = C0 // BJ
GI = R // BI


def _select_body(xt_ref, table_ref, out_ref):
    xb = xt_ref[...][None, :, :]
    t = table_ref[...]
    is1 = xb == 1
    is3 = xb == 3
    islo = xb < 2
    for d0 in range(0, D, DD):
        c0 = t[0, d0:d0 + DD][:, None, None]
        c1 = t[1, d0:d0 + DD][:, None, None]
        c2 = t[2, d0:d0 + DD][:, None, None]
        c3 = t[3, d0:d0 + DD][:, None, None]
        lo = jnp.where(is1, c1, c0)
        hi = jnp.where(is3, c3, c2)
        out_ref[d0:d0 + DD] = jnp.where(islo, lo, hi)


def kernel(x, table):
    xt = jnp.transpose(x.astype(jnp.int32), (1, 0))
    o3 = pl.pallas_call(
        _select_body,
        grid=(GJ, GI),
        in_specs=[
            pl.BlockSpec((BJ, BI), lambda j, i: (j, i)),
            pl.BlockSpec((V, D), lambda j, i: (0, 0)),
        ],
        out_specs=pl.BlockSpec((D, BJ, BI), lambda j, i: (0, j, i)),
        out_shape=jax.ShapeDtypeStruct((D, C0, R), jnp.float32),
    )(xt, table.astype(jnp.float32))
    return jnp.transpose(o3, (2, 1, 0))
